# Initial kernel scaffold; baseline (speedup 1.0000x reference)
import jax
import jax.numpy as jnp
from jax.experimental import pallas as pl


def kernel(atom_x, fg_x, bond_x, ef_edge_index, eb_edge_index, morgan_fp, params):
    raise NotImplementedError("write your pallas kernel here")


# speedup vs baseline: 157.2446x; 157.2446x over previous
import functools

import jax
import jax.numpy as jnp
from jax import lax
from jax.experimental import pallas as pl
from jax.experimental.pallas import tpu as pltpu
from jax.experimental.pallas import tpu_sc as plsc

NA, NF, NB = 10000, 2000, 16000
E = 320000
H, C, HC = 4, 32, 128
F32 = jnp.float32

ET = E // 4
CH = 3200
CROWS = CH // 128
NCHUNK = ET // CH
NSM = 16384
SH = 112640
ZSL = SH // 16


def _prologue_body(ax, fx, bx,
                   fgWs, fgWd, faWs, faWd, bdWs, bdWd, baWs, baWd,
                   fgas, fgad, faas, faad, bdas, bdad, baas, baad,
                   sa_o, sf_o, sb_o, bm_o):
    sel = (lax.broadcasted_iota(jnp.int32, (HC, H), 0) // C ==
           lax.broadcasted_iota(jnp.int32, (HC, H), 1)).astype(F32)

    def proj(w, a):
        return jnp.dot(w[...] * a[...], sel, preferred_element_type=F32)

    def scores(p, x):
        return lax.dot_general(p, x[...], (((0,), (1,)), ((), ())),
                               preferred_element_type=F32)

    pa = jnp.concatenate([proj(fgWs, fgas), proj(faWd, faad),
                          proj(bdWs, bdas), proj(baWd, baad)], axis=1)
    sa = scores(pa, ax)
    sa_o[...] = sa
    pf = jnp.concatenate([proj(fgWd, fgad), proj(faWs, faas)], axis=1)
    sf = scores(pf, fx)
    sf_o[...] = sf
    pb = jnp.concatenate([proj(bdWd, bdad), proj(baWs, baas)], axis=1)
    sb = scores(pb, bx)
    sb_o[...] = sb

    ma = jnp.max(sa, axis=1, keepdims=True)
    mf = jnp.max(sf, axis=1, keepdims=True)
    mb = jnp.max(sb, axis=1, keepdims=True)
    bm = jnp.concatenate([ma[0:4] + mf[0:4], mf[4:8] + ma[4:8],
                          ma[8:12] + mb[0:4], mb[4:8] + ma[12:16]], axis=0)
    bm_o[...] = jnp.maximum(bm, 0.2 * bm) * jnp.ones((1, 16), F32)


def _prologue(ax, fx, bx, ws, avs):
    return pl.pallas_call(
        _prologue_body,
        out_shape=[
            jax.ShapeDtypeStruct((16, NA), F32),
            jax.ShapeDtypeStruct((8, NF), F32),
            jax.ShapeDtypeStruct((8, NB), F32),
            jax.ShapeDtypeStruct((16, 16), F32),
        ],
    )(ax, fx, bx, *ws, *avs)


def _sc_body(sa, sf, sb, bvh, efs, efd, ebs, ebd,
             c1, c2, c3, c4,
             s_src_v, s_dst_v, den_v, esrc_v, edst_v, exv, idxv, zv, bv,
             den_sh, c_sh):
    cid = lax.axis_index("c")
    sid = lax.axis_index("s")
    h = sid // 4
    q = sid % 4
    @pl.loop(0, ZSL // 16)
    def _zero(i):
        zv[pl.ds(i * 16, 16)] = jnp.zeros((16,), F32)

    pltpu.sync_copy(zv, den_sh.at[pl.ds(sid * ZSL, ZSL)])
    pltpu.sync_copy(zv, c_sh.at[pl.ds(sid * ZSL, ZSL)])
    plsc.subcore_barrier()

    g1 = dict(sat=sa, srow=0, dat=sf, drow=0, ns=NA, nd=NF,
              esrc=efs, edst=efd, den_off=0, c_off=0, bbase=0,
              cout=c1, csz=4 * NA)
    g2 = dict(sat=sf, srow=4, dat=sa, drow=4, ns=NF, nd=NA,
              esrc=efd, edst=efs, den_off=8000, c_off=40000, bbase=4,
              cout=c2, csz=4 * NF)
    g3 = dict(sat=sa, srow=8, dat=sb, drow=0, ns=NA, nd=NB,
              esrc=ebs, edst=ebd, den_off=0, c_off=0, bbase=8,
              cout=c3, csz=4 * NA)
    g4 = dict(sat=sb, srow=4, dat=sa, drow=12, ns=NB, nd=NA,
              esrc=ebd, edst=ebs, den_off=64000, c_off=40000, bbase=12,
              cout=c4, csz=4 * NB)

    def load_tables(cfg):
        pltpu.sync_copy(cfg["sat"].at[pl.ds((cfg["srow"] + h) * cfg["ns"],
                                            cfg["ns"])],
                        s_src_v.at[pl.ds(0, cfg["ns"])])
        pltpu.sync_copy(cfg["dat"].at[pl.ds((cfg["drow"] + h) * cfg["nd"],
                                            cfg["nd"])],
                        s_dst_v.at[pl.ds(0, cfg["nd"])])

    def edge_pass(cfg, second):
        load_tables(cfg)
        pltpu.sync_copy(bvh.at[pl.ds((cfg["bbase"] + h) * 16, 16)],
                        bv.at[pl.ds(0, 16)])
        bvec = bv[pl.ds(0, 16)]
        doff = cfg["den_off"] + h * cfg["nd"]
        coff = cfg["c_off"] + h * cfg["ns"]
        if second:
            pltpu.sync_copy(den_sh.at[pl.ds(doff, cfg["nd"])],
                            den_v.at[pl.ds(0, cfg["nd"])])

        @pl.loop(0, NCHUNK)
        def _chunk(ci):
            e0 = q * ET + ci * CH
            pltpu.sync_copy(cfg["esrc"].at[pl.ds(e0, CH)], esrc_v)
            pltpu.sync_copy(cfg["edst"].at[pl.ds(e0, CH)], edst_v)

            @pl.loop(0, CROWS)
            def _row(vi):
                for j in range(8):
                    fl = pl.ds(vi * 128 + j * 16, 16)
                    si = esrc_v[fl]
                    di = edst_v[fl]
                    a = plsc.load_gather(s_src_v, [si])
                    b = plsc.load_gather(s_dst_v, [di])
                    logit = a + b
                    logit = jnp.maximum(logit, logit * 0.2)
                    ex = jnp.exp(logit - bvec)
                    if second:
                        dv = plsc.load_gather(den_v, [di])
                        exv[fl] = ex / (dv + 1e-16)
                        idxv[vi, pl.ds(j * 16, 16)] = si + coff
                    else:
                        exv[fl] = ex
                        idxv[vi, pl.ds(j * 16, 16)] = di + doff

            acc = c_sh if second else den_sh

            @pl.loop(0, CROWS)
            def _scat(vi):
                pltpu.sync_copy(exv.at[pl.ds(vi * 128, 128)],
                                acc.at[idxv.at[vi]], add=True)

    @pl.when(cid == 0)
    def _p1a():
        edge_pass(g1, False)
        edge_pass(g2, False)

    @pl.when(cid == 1)
    def _p1b():
        edge_pass(g3, False)
        edge_pass(g4, False)

    plsc.subcore_barrier()

    @pl.when(cid == 0)
    def _p2a():
        edge_pass(g1, True)
        edge_pass(g2, True)

    @pl.when(cid == 1)
    def _p2b():
        edge_pass(g3, True)
        edge_pass(g4, True)

    plsc.subcore_barrier()

    def copy_out(cfg):
        per = 4000
        ntile = cfg["csz"] // per

        @pl.when(sid < ntile)
        def _cp():
            off = sid * per
            pltpu.sync_copy(c_sh.at[pl.ds(cfg["c_off"] + off, per)],
                            zv.at[pl.ds(0, per)])
            pltpu.sync_copy(zv.at[pl.ds(0, per)],
                            cfg["cout"].at[pl.ds(off, per)])

    @pl.when(cid == 0)
    def _oa():
        copy_out(g1)
        copy_out(g2)

    @pl.when(cid == 1)
    def _ob():
        copy_out(g3)
        copy_out(g4)


def _sc_gat(sa, sf, sb, bvec, efs, efd, ebs, ebd):
    mesh = plsc.VectorSubcoreMesh(core_axis_name="c", subcore_axis_name="s",
                                  num_cores=2, num_subcores=16)
    fn = pl.kernel(
        _sc_body,
        compiler_params=pltpu.CompilerParams(needs_layout_passes=False),
        out_type=[
            jax.ShapeDtypeStruct((4 * NA,), F32),
            jax.ShapeDtypeStruct((4 * NF,), F32),
            jax.ShapeDtypeStruct((4 * NA,), F32),
            jax.ShapeDtypeStruct((4 * NB,), F32),
        ],
        mesh=mesh,
        scratch_types=[
            pltpu.VMEM((NSM,), F32),
            pltpu.VMEM((NSM,), F32),
            pltpu.VMEM((NSM,), F32),
            pltpu.VMEM((CH,), jnp.int32),
            pltpu.VMEM((CH,), jnp.int32),
            pltpu.VMEM((CH,), F32),
            pltpu.VMEM((CROWS, 128), jnp.int32),
            pltpu.VMEM((ZSL,), F32),
            pltpu.VMEM((128,), F32),
            pltpu.VMEM_SHARED((SH,), F32),
            pltpu.VMEM_SHARED((SH,), F32),
        ],
    )
    return fn(sa, sf, sb, bvec, efs, efd, ebs, ebd)


def _tail_body(c1, c2, c3, c4, ax, fx, bx,
               fgWs, faWs, bdWs, baWs,
               wp, bp, wg1, wg2, bg, morgan,
               w1, b1, w2, b2, w3, b3, out):
    mask = (lax.broadcasted_iota(jnp.int32, (H, HC), 1) // C ==
            lax.broadcasted_iota(jnp.int32, (H, HC), 0)).astype(F32)

    def head_vec(cw, x, w):
        m = lax.dot_general(cw[...], x[...], (((1,), (0,)), ((), ())),
                            preferred_element_type=F32)
        g = jnp.dot(m, w[...], preferred_element_type=F32)
        return jnp.sum(g * mask, axis=0, keepdims=True)

    v1 = head_vec(c1, ax, fgWs)
    v2 = head_vec(c2, fx, faWs)
    v3 = head_vec(c3, ax, bdWs)
    v4 = head_vec(c4, bx, baWs)
    fused = (v1 * (1.0 / NF) + v2 * (1.0 / NA) +
             v3 * (1.0 / NB) + v4 * (1.0 / NA)) * 0.25

    proj = jnp.dot(fused, wp[...], preferred_element_type=F32) + bp[...]
    gsc = (jnp.dot(proj, wg1[...], preferred_element_type=F32) +
           jnp.dot(morgan[...], wg2[...], preferred_element_type=F32) +
           bg[...])
    gate = 1.0 / (1.0 + jnp.exp(-gsc))
    enh = gate * proj + (1.0 - gate) * morgan[...]
    h1 = jnp.maximum(jnp.dot(enh, w1[...], preferred_element_type=F32)
                     + b1[...], 0.0)
    h2 = jnp.maximum(jnp.dot(h1, w2[...], preferred_element_type=F32)
                     + b2[...], 0.0)
    out[...] = jnp.dot(h2, w3[...], preferred_element_type=F32) + b3[...]


def _tail(*args):
    return pl.pallas_call(
        _tail_body,
        out_shape=jax.ShapeDtypeStruct((1, 1), F32),
    )(*args)


def kernel(atom_x, fg_x, bond_x, ef_edge_index, eb_edge_index, morgan_fp,
           params):
    p = params
    ws = (p["fg_Ws"], p["fg_Wd"], p["fa_Ws"], p["fa_Wd"],
          p["bd_Ws"], p["bd_Wd"], p["ba_Ws"], p["ba_Wd"])
    avs = tuple(p[k].reshape(1, HC) for k in
                ("fg_as", "fg_ad", "fa_as", "fa_ad",
                 "bd_as", "bd_ad", "ba_as", "ba_ad"))

    sa, sf, sb, bm = _prologue(atom_x, fg_x, bond_x, ws, avs)

    i32 = jnp.int32
    efs = ef_edge_index[0].astype(i32)
    efd = ef_edge_index[1].astype(i32)
    ebs = eb_edge_index[0].astype(i32)
    ebd = eb_edge_index[1].astype(i32)

    c1, c2, c3, c4 = _sc_gat(sa.reshape(16 * NA), sf.reshape(8 * NF),
                             sb.reshape(8 * NB), bm.reshape(256),
                             efs, efd, ebs, ebd)

    out = _tail(c1.reshape(4, NA), c2.reshape(4, NF),
                c3.reshape(4, NA), c4.reshape(4, NB),
                atom_x, fg_x, bond_x,
                p["fg_Ws"], p["fa_Ws"], p["bd_Ws"], p["ba_Ws"],
                p["Wp"], p["bp"].reshape(1, -1),
                p["Wg"][:2048], p["Wg"][2048:], p["bg"].reshape(1, 1),
                morgan_fp.reshape(1, -1),
                p["W1"], p["b1"].reshape(1, -1),
                p["W2"], p["b2"].reshape(1, -1),
                p["W3"], p["b3"].reshape(1, 1))
    return out.reshape(1)
